# baseline (speedup 1.0000x reference)
import functools

import jax
import jax.numpy as jnp
from jax import lax
from jax.experimental import pallas as pl
from jax.experimental.pallas import tpu as pltpu
from jax.experimental.pallas import tpu_sc as plsc

_G = 2
_V = 320
_D = 256
_TILE = 512


def _vq_tc_body(x_ref, w_ref, b_ref, idx_ref, perp_ref, acc_ref):
    i = pl.program_id(0)
    n = pl.num_programs(0)

    x = x_ref[...]
    logits = lax.dot_general(
        x, w_ref[...], (((1,), (1,)), ((), ()))
    ) + b_ref[...]

    iota = lax.broadcasted_iota(jnp.int32, (_TILE, _V), 1)
    idxs = []
    counts = []
    for g in range(_G):
        lg = logits[:, g * _V:(g + 1) * _V]
        m = jnp.max(lg, axis=1, keepdims=True)
        cand = jnp.where(lg == m, iota, _V)
        idx = jnp.min(cand, axis=1, keepdims=True)
        oh = (iota == idx).astype(jnp.float32)
        idxs.append(idx + g * _V)
        counts.append(jnp.sum(oh, axis=0, keepdims=True))

    idx_ref[...] = jnp.concatenate(idxs, axis=1)

    @pl.when(i == 0)
    def _():
        acc_ref[...] = jnp.zeros_like(acc_ref)

    acc_ref[...] += jnp.concatenate(counts, axis=0)

    @pl.when(i == n - 1)
    def _():
        p = acc_ref[...] / jnp.float32(n * _TILE)
        ent = jnp.sum(p * jnp.log(p + 1e-7), axis=1, keepdims=True)
        perp_ref[...] = jnp.sum(jnp.exp(-ent), keepdims=True)


_INFO = plsc.get_sparse_core_info()
_NC = _INFO.num_cores
_NS = _INFO.num_subcores
_NW = _NC * _NS
_CHUNK = 128


def _sc_gather_body(idx_hbm, table_hbm, out_hbm, idx_v, rows0_v, rows1_v,
                    sem0, sem1):
    rows_total = out_hbm.shape[0]
    rows_per_w = rows_total // _NW
    wid = lax.axis_index("s") * _NC + lax.axis_index("c")
    base = wid * rows_per_w
    pltpu.sync_copy(idx_hbm.at[pl.ds(base, rows_per_w)], idx_v)

    nchunks = rows_per_w // _CHUNK
    bufs = (rows0_v, rows1_v)
    sems = (sem0, sem1)
    copies = [None, None]
    for c in range(nchunks):
        s = c % 2
        copies[s] = pltpu.async_copy(
            table_hbm.at[idx_v.at[pl.ds(c * _CHUNK, _CHUNK)]], bufs[s],
            sems[s])
        if c > 0:
            copies[1 - s].wait()
            pltpu.sync_copy(
                bufs[1 - s], out_hbm.at[pl.ds(base + (c - 1) * _CHUNK,
                                              _CHUNK)])
    last = (nchunks - 1) % 2
    copies[last].wait()
    pltpu.sync_copy(
        bufs[last], out_hbm.at[pl.ds(base + (nchunks - 1) * _CHUNK, _CHUNK)])


def kernel(hidden_states, W, b, codevectors):
    B, S, H = hidden_states.shape
    N = B * S
    x = hidden_states.reshape(N, H)
    cv = codevectors.reshape(_G * _V, _D)
    b2 = b.reshape(1, _G * _V)

    idx2d, perp = pl.pallas_call(
        _vq_tc_body,
        grid=(N // _TILE,),
        in_specs=[
            pl.BlockSpec((_TILE, H), lambda i: (i, 0)),
            pl.BlockSpec((_G * _V, H), lambda i: (0, 0)),
            pl.BlockSpec((1, _G * _V), lambda i: (0, 0)),
        ],
        out_specs=[
            pl.BlockSpec((_TILE, _G), lambda i: (i, 0)),
            pl.BlockSpec((1, 1), lambda i: (0, 0)),
        ],
        out_shape=[
            jax.ShapeDtypeStruct((N, _G), jnp.int32),
            jax.ShapeDtypeStruct((1, 1), jnp.float32),
        ],
        scratch_shapes=[pltpu.VMEM((_G, _V), jnp.float32)],
        compiler_params=pltpu.CompilerParams(
            dimension_semantics=("arbitrary",)
        ),
    )(x, W, b2)

    idx_flat = idx2d.reshape(N * _G)

    sc_gather = functools.partial(
        pl.kernel,
        mesh=plsc.VectorSubcoreMesh(core_axis_name="c", subcore_axis_name="s"),
        out_type=jax.ShapeDtypeStruct((N * _G, _D), jnp.float32),
        scratch_types=[
            pltpu.VMEM((N * _G // _NW,), jnp.int32),
            pltpu.VMEM((_CHUNK, _D), jnp.float32),
            pltpu.VMEM((_CHUNK, _D), jnp.float32),
            pltpu.SemaphoreType.DMA,
            pltpu.SemaphoreType.DMA,
        ],
    )(_sc_gather_body)

    rows = sc_gather(idx_flat, cv)
    return rows.reshape(B, S, _G * _D), perp[0, 0]


# speedup vs baseline: 2.3934x; 2.3934x over previous
import jax
import jax.numpy as jnp
from jax import lax
from jax.experimental import pallas as pl
from jax.experimental.pallas import tpu as pltpu

_G = 2
_V = 320
_D = 256
_TILE = 512


def _vq_body(x_ref, w_ref, b_ref, cv_ref, out_ref, perp_ref, acc_ref):
    i = pl.program_id(0)
    n = pl.num_programs(0)

    x = x_ref[...]
    logits = lax.dot_general(
        x, w_ref[...], (((1,), (1,)), ((), ()))
    ) + b_ref[...]

    iota = lax.broadcasted_iota(jnp.int32, (_TILE, _V), 1)
    outs = []
    counts = []
    for g in range(_G):
        lg = logits[:, g * _V:(g + 1) * _V]
        m = jnp.max(lg, axis=1, keepdims=True)
        cand = jnp.where(lg == m, iota, _V)
        idx = jnp.min(cand, axis=1, keepdims=True)
        oh = (iota == idx).astype(jnp.float32)
        cvg = cv_ref[g * _V:(g + 1) * _V, :]
        outs.append(jnp.dot(oh, cvg))
        counts.append(jnp.sum(oh, axis=0, keepdims=True))

    out_ref[...] = jnp.concatenate(outs, axis=1)

    @pl.when(i == 0)
    def _():
        acc_ref[...] = jnp.zeros_like(acc_ref)

    acc_ref[...] += jnp.concatenate(counts, axis=0)

    @pl.when(i == n - 1)
    def _():
        p = acc_ref[...] / jnp.float32(n * _TILE)
        ent = jnp.sum(p * jnp.log(p + 1e-7), axis=1, keepdims=True)
        perp_ref[...] = jnp.sum(jnp.exp(-ent), keepdims=True)


def kernel(hidden_states, W, b, codevectors):
    B, S, H = hidden_states.shape
    N = B * S
    x = hidden_states.reshape(N, H)
    cv = codevectors.reshape(_G * _V, _D)
    b2 = b.reshape(1, _G * _V)

    out, perp = pl.pallas_call(
        _vq_body,
        grid=(N // _TILE,),
        in_specs=[
            pl.BlockSpec((_TILE, H), lambda i: (i, 0)),
            pl.BlockSpec((_G * _V, H), lambda i: (0, 0)),
            pl.BlockSpec((1, _G * _V), lambda i: (0, 0)),
            pl.BlockSpec((_G * _V, _D), lambda i: (0, 0)),
        ],
        out_specs=[
            pl.BlockSpec((_TILE, _G * _D), lambda i: (i, 0)),
            pl.BlockSpec((1, 1), lambda i: (0, 0)),
        ],
        out_shape=[
            jax.ShapeDtypeStruct((N, _G * _D), jnp.float32),
            jax.ShapeDtypeStruct((1, 1), jnp.float32),
        ],
        scratch_shapes=[pltpu.VMEM((_G, _V), jnp.float32)],
        compiler_params=pltpu.CompilerParams(
            dimension_semantics=("arbitrary",)
        ),
    )(x, W, b2, cv)
    return out.reshape(B, S, _G * _D), perp[0, 0]


# speedup vs baseline: 2.4810x; 1.0366x over previous
import jax
import jax.numpy as jnp
from jax import lax
from jax.experimental import pallas as pl
from jax.experimental.pallas import tpu as pltpu

_G = 2
_V = 320
_D = 256
_TILE = 512


def _vq_body(x_ref, w_ref, b_ref, cv_ref, out_ref, perp_ref, acc_ref):
    i = pl.program_id(0)
    n = pl.num_programs(0)

    x = x_ref[...]
    logits = lax.dot_general(
        x, w_ref[...], (((1,), (1,)), ((), ()))
    ) + b_ref[...]

    iota = lax.broadcasted_iota(jnp.int32, (_TILE, _V), 1).astype(jnp.float32)
    outs = []
    counts = []
    for g in range(_G):
        lg = logits[:, g * _V:(g + 1) * _V]
        m = jnp.max(lg, axis=1, keepdims=True)
        cand = jnp.where(lg == m, iota, jnp.float32(_V))
        idx = jnp.min(cand, axis=1, keepdims=True)
        oh = (iota == idx).astype(jnp.float32)
        cvg = cv_ref[g * _V:(g + 1) * _V, :]
        outs.append(jnp.dot(oh, cvg))
        counts.append(jnp.sum(oh, axis=0, keepdims=True))

    out_ref[...] = jnp.concatenate(outs, axis=1)

    @pl.when(i == 0)
    def _():
        acc_ref[...] = jnp.zeros_like(acc_ref)

    acc_ref[...] += jnp.concatenate(counts, axis=0)

    @pl.when(i == n - 1)
    def _():
        p = acc_ref[...] / jnp.float32(n * _TILE)
        ent = jnp.sum(p * jnp.log(p + 1e-7), axis=1, keepdims=True)
        perp_ref[...] = jnp.sum(jnp.exp(-ent), keepdims=True)


def kernel(hidden_states, W, b, codevectors):
    B, S, H = hidden_states.shape
    N = B * S
    x = hidden_states.reshape(N, H)
    cv = codevectors.reshape(_G * _V, _D)
    b2 = b.reshape(1, _G * _V)

    out, perp = pl.pallas_call(
        _vq_body,
        grid=(N // _TILE,),
        in_specs=[
            pl.BlockSpec((_TILE, H), lambda i: (i, 0)),
            pl.BlockSpec((_G * _V, H), lambda i: (0, 0)),
            pl.BlockSpec((1, _G * _V), lambda i: (0, 0)),
            pl.BlockSpec((_G * _V, _D), lambda i: (0, 0)),
        ],
        out_specs=[
            pl.BlockSpec((_TILE, _G * _D), lambda i: (i, 0)),
            pl.BlockSpec((1, 1), lambda i: (0, 0)),
        ],
        out_shape=[
            jax.ShapeDtypeStruct((N, _G * _D), jnp.float32),
            jax.ShapeDtypeStruct((1, 1), jnp.float32),
        ],
        scratch_shapes=[pltpu.VMEM((_G, _V), jnp.float32)],
        compiler_params=pltpu.CompilerParams(
            dimension_semantics=("arbitrary",)
        ),
    )(x, W, b2, cv)
    return out.reshape(B, S, _G * _D), perp[0, 0]


# speedup vs baseline: 2.8863x; 1.1634x over previous
import jax
import jax.numpy as jnp
from jax import lax
from jax.experimental import pallas as pl
from jax.experimental.pallas import tpu as pltpu

_G = 2
_V = 320
_D = 256
_TILE = 1024


def _vq_body(x_ref, w_ref, b_ref, cv_ref, out_ref, perp_ref, acc_ref):
    i = pl.program_id(0)
    n = pl.num_programs(0)

    x = x_ref[...]
    logits = lax.dot_general(
        x, w_ref[...], (((1,), (1,)), ((), ()))
    ) + b_ref[...]

    iota = lax.broadcasted_iota(jnp.int32, (_TILE, _V), 1).astype(jnp.float32)
    outs = []
    counts = []
    for g in range(_G):
        lg = logits[:, g * _V:(g + 1) * _V]
        m = jnp.max(lg, axis=1, keepdims=True)
        cand = jnp.where(lg == m, iota, jnp.float32(_V))
        idx = jnp.min(cand, axis=1, keepdims=True)
        oh = (iota == idx).astype(jnp.float32)
        cvg = cv_ref[g * _V:(g + 1) * _V, :]
        outs.append(jnp.dot(oh, cvg))
        counts.append(jnp.sum(oh, axis=0, keepdims=True))

    out_ref[...] = jnp.concatenate(outs, axis=1)

    @pl.when(i == 0)
    def _():
        acc_ref[...] = jnp.zeros_like(acc_ref)

    acc_ref[...] += jnp.concatenate(counts, axis=0)

    @pl.when(i == n - 1)
    def _():
        p = acc_ref[...] / jnp.float32(n * _TILE)
        ent = jnp.sum(p * jnp.log(p + 1e-7), axis=1, keepdims=True)
        perp_ref[...] = jnp.sum(jnp.exp(-ent), keepdims=True)


def kernel(hidden_states, W, b, codevectors):
    B, S, H = hidden_states.shape
    N = B * S
    x = hidden_states.reshape(N, H)
    cv = codevectors.reshape(_G * _V, _D)
    b2 = b.reshape(1, _G * _V)

    out, perp = pl.pallas_call(
        _vq_body,
        grid=(N // _TILE,),
        in_specs=[
            pl.BlockSpec((_TILE, H), lambda i: (i, 0)),
            pl.BlockSpec((_G * _V, H), lambda i: (0, 0)),
            pl.BlockSpec((1, _G * _V), lambda i: (0, 0)),
            pl.BlockSpec((_G * _V, _D), lambda i: (0, 0)),
        ],
        out_specs=[
            pl.BlockSpec((_TILE, _G * _D), lambda i: (i, 0)),
            pl.BlockSpec((1, 1), lambda i: (0, 0)),
        ],
        out_shape=[
            jax.ShapeDtypeStruct((N, _G * _D), jnp.float32),
            jax.ShapeDtypeStruct((1, 1), jnp.float32),
        ],
        scratch_shapes=[pltpu.VMEM((_G, _V), jnp.float32)],
        compiler_params=pltpu.CompilerParams(
            dimension_semantics=("arbitrary",)
        ),
    )(x, W, b2, cv)
    return out.reshape(B, S, _G * _D), perp[0, 0]
